# baseline (speedup 1.0000x reference)
import jax
import jax.numpy as jnp
from jax.experimental import pallas as pl
from jax.experimental.pallas import tpu as pltpu
from jax.experimental.pallas import tpu_sc as plsc

_LANES = 16


def kernel(dense_features, sparse_features, emb_weight, bot_w, top_w):
    n_valid = sparse_features.shape[0]
    n_rows, emb_dim = emb_weight.shape
    pad_len = -(-n_valid // _LANES) * _LANES

    par = jnp.concatenate([
        emb_weight.reshape(-1),
        dense_features.reshape(-1),
        bot_w.reshape(-1),
        top_w.reshape(-1),
    ]).astype(jnp.float32)
    n_par = -(-par.shape[0] // _LANES) * _LANES
    packed = jnp.concatenate([
        sparse_features.astype(jnp.int32),
        jnp.zeros((pad_len - n_valid,), jnp.int32),
        jax.lax.bitcast_convert_type(
            jnp.pad(par, (0, n_par - par.shape[0])), jnp.int32),
    ])

    mesh = plsc.VectorSubcoreMesh(
        core_axis_name="c", subcore_axis_name="s", num_cores=1, num_subcores=1)

    def body(packed_hbm, out_hbm, buf_v, out_v):
        pltpu.sync_copy(packed_hbm, buf_v)

        def step(i, carry):
            a1, a2 = carry
            v = buf_v[pl.ds(i * _LANES, _LANES)]
            return a1 + v, a2 + v * v

        zero = jnp.zeros((_LANES,), jnp.int32)
        a1, a2 = jax.lax.fori_loop(0, pad_len // _LANES, step, (zero, zero))
        s1 = jnp.sum(a1).astype(jnp.float32)
        s2 = jnp.sum(a2).astype(jnp.float32)
        c2 = (s2 - s1) * 0.5
        c1 = 2.0 * s1 - s2
        counts = [float(n_valid) - c1 - c2, c1, c2]

        pv = plsc.bitcast(buf_v[pl.ds(pad_len, _LANES)], jnp.float32)

        def p(k):
            return pv[k]

        e_base = 0
        d_base = n_rows * emb_dim
        b_base = d_base + emb_dim
        t_base = b_base + 4

        scale = 2.0 / float(n_valid)
        y = [
            sum(counts[r] * p(e_base + r * emb_dim + c) for r in range(n_rows))
            * scale
            for c in range(emb_dim)
        ]
        d = [p(d_base + k) for k in range(emb_dim)]
        x = [sum(d[k] * p(b_base + j * 2 + k) for k in range(2)) for j in range(2)]
        z = x + y
        out = sum(z[j] * p(t_base + j) for j in range(4))

        out_v[...] = out * jnp.ones((_LANES,), jnp.float32)
        pltpu.sync_copy(out_v, out_hbm)

    out16 = pl.kernel(
        body,
        out_type=jax.ShapeDtypeStruct((_LANES,), jnp.float32),
        mesh=mesh,
        compiler_params=pltpu.CompilerParams(needs_layout_passes=False),
        scratch_types=[
            pltpu.VMEM((pad_len + n_par,), jnp.int32),
            pltpu.VMEM((_LANES,), jnp.float32),
        ],
    )(packed)

    return out16[:1].reshape(1, 1)


# speedup vs baseline: 1.1112x; 1.1112x over previous
import jax
import jax.numpy as jnp
from jax.experimental import pallas as pl
from jax.experimental.pallas import tpu as pltpu
from jax.experimental.pallas import tpu_sc as plsc

_UNROLL = 8


def kernel(dense_features, sparse_features, emb_weight, bot_w, top_w):
    n_valid = sparse_features.shape[0]
    n_rows, emb_dim = emb_weight.shape
    idx = sparse_features.astype(jnp.int32)

    mesh = plsc.ScalarSubcoreMesh(axis_name="c", num_cores=1)

    def body(idx_hbm, dense_hbm, emb_hbm, bot_hbm, top_hbm, out_hbm,
             idx_s, dense_s, emb_s, bot_s, top_s, out_s, sem):
        copies = [
            pltpu.make_async_copy(idx_hbm, idx_s, sem),
            pltpu.make_async_copy(dense_hbm, dense_s, sem),
            pltpu.make_async_copy(emb_hbm, emb_s, sem),
            pltpu.make_async_copy(bot_hbm, bot_s, sem),
            pltpu.make_async_copy(top_hbm, top_s, sem),
        ]
        for c in copies:
            c.start()
        for c in copies:
            c.wait()

        def step(i, carry):
            s1, s2 = carry
            for u in range(_UNROLL):
                v = idx_s[i * _UNROLL + u]
                s1 = s1 + v
                s2 = s2 + v * v
            return s1, s2

        s1i, s2i = jax.lax.fori_loop(
            0, n_valid // _UNROLL, step, (jnp.int32(0), jnp.int32(0)))
        for u in range(n_valid - (n_valid // _UNROLL) * _UNROLL):
            v = idx_s[(n_valid // _UNROLL) * _UNROLL + u]
            s1i = s1i + v
            s2i = s2i + v * v
        s1 = s1i.astype(jnp.float32)
        s2 = s2i.astype(jnp.float32)
        c2 = (s2 - s1) * 0.5
        c1 = 2.0 * s1 - s2
        counts = [float(n_valid) - c1 - c2, c1, c2]

        scale = 2.0 / float(n_valid)
        y = [
            sum(counts[r] * emb_s[r, c] for r in range(n_rows)) * scale
            for c in range(emb_dim)
        ]
        d = [dense_s[0, k] for k in range(emb_dim)]
        x = [sum(d[k] * bot_s[j, k] for k in range(2)) for j in range(2)]
        z = x + y
        out = sum(z[j] * top_s[0, j] for j in range(4))

        out_s[0, 0] = out
        pltpu.sync_copy(out_s, out_hbm)

    return pl.kernel(
        body,
        out_type=jax.ShapeDtypeStruct((1, 1), jnp.float32),
        mesh=mesh,
        compiler_params=pltpu.CompilerParams(needs_layout_passes=False),
        scratch_types=[
            pltpu.SMEM((n_valid,), jnp.int32),
            pltpu.SMEM((1, emb_dim), jnp.float32),
            pltpu.SMEM((n_rows, emb_dim), jnp.float32),
            pltpu.SMEM((2, 2), jnp.float32),
            pltpu.SMEM((1, 4), jnp.float32),
            pltpu.SMEM((1, 1), jnp.float32),
            pltpu.SemaphoreType.DMA,
        ],
    )(idx, dense_features, emb_weight, bot_w, top_w)
